# Initial kernel scaffold; baseline (speedup 1.0000x reference)
import jax
import jax.numpy as jnp
from jax.experimental import pallas as pl


def kernel(x, edge_index, W1l, W1r, b1, W2l, W2r, b2, W3l, W3r, b3):
    raise NotImplementedError("write your pallas kernel here")


# speedup vs baseline: 2.7780x; 2.7780x over previous
import jax
import jax.numpy as jnp
from jax import lax
from jax.experimental import pallas as pl
from jax.experimental.pallas import tpu as pltpu
from jax.experimental.pallas import tpu_sc as plsc

N_NODES = 10000
N_EDGES = 160000
D = 256
DH = 128

NUM_TILES = 16
CHUNK = 128
CHUNKS_PER_TILE = 80
GCH = 8
GROUPS = CHUNKS_PER_TILE // GCH
EDGES_PER_TILE = CHUNK * CHUNKS_PER_TILE
E_PAD = EDGES_PER_TILE * NUM_TILES
N_PAD = 10112
ROWS_PER_TILE = N_PAD // NUM_TILES

def _mesh():
  return plsc.VectorSubcoreMesh(core_axis_name="c", subcore_axis_name="s")


def _sc_aggregate(h2, src4, dst3, zf):
  def body(h2_hbm, src_hbm, dst_hbm, zf_hbm, agg_hbm,
           accum, sbuf, dbuf, rows, sem):
    c = lax.axis_index("c")
    t = lax.axis_index("s")
    row0 = t * ROWS_PER_TILE
    pltpu.sync_copy(zf_hbm, accum.at[pl.ds(row0, ROWS_PER_TILE)])
    plsc.subcore_barrier()

    def group(g, carry):
      pltpu.sync_copy(src_hbm.at[c, t * GROUPS + g], sbuf)
      pltpu.sync_copy(dst_hbm.at[t * GROUPS + g], dbuf)

      def chunk(j, carry2):
        pltpu.async_copy(h2_hbm.at[sbuf.at[j]], rows, sem).wait()
        pltpu.sync_copy(rows, accum.at[dbuf.at[j]], add=True)
        return carry2

      lax.fori_loop(0, GCH, chunk, 0)
      return carry

    lax.fori_loop(0, GROUPS, group, 0)
    plsc.subcore_barrier()
    pltpu.sync_copy(accum.at[pl.ds(row0, ROWS_PER_TILE)],
                    agg_hbm.at[c, pl.ds(row0, ROWS_PER_TILE)])

  fn = pl.kernel(
      body,
      out_type=jax.ShapeDtypeStruct((2, N_PAD, DH), jnp.float32),
      mesh=_mesh(),
      scratch_types=(pltpu.VMEM_SHARED((N_PAD, DH), jnp.float32),
                     pltpu.VMEM((GCH, CHUNK), jnp.int32),
                     pltpu.VMEM((GCH, CHUNK), jnp.int32),
                     pltpu.VMEM((CHUNK, DH), jnp.float32),
                     pltpu.SemaphoreType.DMA))
  return fn(h2, src4, dst3, zf)


def _sc_count(dst3, zf, ones128):
  half = GROUPS // 2

  def body(dst_hbm, zf_hbm, ones_hbm, cnt_hbm, accum, dbuf, ones_v, sem):
    c = lax.axis_index("c")
    t = lax.axis_index("s")
    row0 = t * ROWS_PER_TILE
    pltpu.sync_copy(zf_hbm, accum.at[pl.ds(row0, ROWS_PER_TILE)])
    pltpu.sync_copy(ones_hbm, ones_v)
    plsc.subcore_barrier()

    def group(g, carry):
      pltpu.sync_copy(dst_hbm.at[t * GROUPS + c * half + g], dbuf)

      def chunk(j, carry2):
        pltpu.sync_copy(ones_v, accum.at[dbuf.at[j]], add=True)
        return carry2

      lax.fori_loop(0, GCH, chunk, 0)
      return carry

    lax.fori_loop(0, half, group, 0)
    plsc.subcore_barrier()
    pltpu.sync_copy(accum.at[pl.ds(row0, ROWS_PER_TILE)],
                    cnt_hbm.at[c, pl.ds(row0, ROWS_PER_TILE)])

  fn = pl.kernel(
      body,
      out_type=jax.ShapeDtypeStruct((2, N_PAD, DH), jnp.float32),
      mesh=_mesh(),
      scratch_types=(pltpu.VMEM_SHARED((N_PAD, DH), jnp.float32),
                     pltpu.VMEM((GCH, CHUNK), jnp.int32),
                     pltpu.VMEM((CHUNK, DH), jnp.float32),
                     pltpu.SemaphoreType.DMA))
  return fn(dst3, zf, ones128)


ROW_BLK = 400
GRID = N_NODES // ROW_BLK


def _tc_body(agg_ref, cnt_ref, h_ref, wl_ref, wr_ref, b_ref, *out_refs):
  agg = jnp.concatenate([agg_ref[0], agg_ref[1]], axis=1)
  cnt = cnt_ref[0][:, 0:1] + cnt_ref[1][:, 0:1]
  inv = 1.0 / jnp.maximum(cnt, 1.0)
  h = jnp.concatenate([h_ref[0], h_ref[1]], axis=1)
  out = (jnp.dot(agg * inv, wl_ref[...], preferred_element_type=jnp.float32)
         + jnp.dot(h, wr_ref[...], preferred_element_type=jnp.float32)
         + b_ref[...])
  act = jnp.where(out > 0, out, jnp.exp(jnp.minimum(out, 0.0)) - 1.0)
  if len(out_refs) == 1 and out_refs[0].shape[0] == 2:
    out_refs[0][0] = act[:, :DH]
    out_refs[0][1] = act[:, DH:]
  else:
    out_refs[0][...] = act


def _tc_layer(agg, cnt, h2, wl, wr, b, split_out):
  if split_out:
    out_shape = [jax.ShapeDtypeStruct((2, N_NODES, DH), jnp.float32)]
    out_specs = [pl.BlockSpec((2, ROW_BLK, DH), lambda r: (0, r, 0))]
  else:
    out_shape = [jax.ShapeDtypeStruct((N_NODES, D), jnp.float32)]
    out_specs = [pl.BlockSpec((ROW_BLK, D), lambda r: (r, 0))]
  res = pl.pallas_call(
      _tc_body,
      grid=(GRID,),
      in_specs=[
          pl.BlockSpec((2, ROW_BLK, DH), lambda r: (0, r, 0)),
          pl.BlockSpec((2, ROW_BLK, DH), lambda r: (0, r, 0)),
          pl.BlockSpec((2, ROW_BLK, DH), lambda r: (0, r, 0)),
          pl.BlockSpec((D, D), lambda r: (0, 0)),
          pl.BlockSpec((D, D), lambda r: (0, 0)),
          pl.BlockSpec((1, D), lambda r: (0, 0)),
      ],
      out_specs=out_specs,
      out_shape=out_shape,
  )(agg, cnt, h2, wl, wr, b)
  return res[0]


def kernel(x, edge_index, W1l, W1r, b1, W2l, W2r, b2, W3l, W3r, b3):
  x2 = jnp.stack([x[:, :DH], x[:, DH:]])
  src = edge_index[0]
  dst = edge_index[1]
  pad = E_PAD - N_EDGES
  src_p = jnp.concatenate([src, jnp.zeros((pad,), jnp.int32)])
  src4 = jnp.stack([src_p, src_p + N_NODES]).reshape(
      2, NUM_TILES * GROUPS, GCH, CHUNK)
  dst3 = jnp.concatenate([dst, jnp.full((pad,), N_NODES, jnp.int32)]).reshape(
      NUM_TILES * GROUPS, GCH, CHUNK)
  zf = jnp.zeros((ROWS_PER_TILE, DH), jnp.float32)
  ones128 = jnp.ones((CHUNK, DH), jnp.float32)

  cnt = _sc_count(dst3, zf, ones128)
  h2 = x2
  agg = _sc_aggregate(h2.reshape(2 * N_NODES, DH), src4, dst3, zf)
  h2 = _tc_layer(agg, cnt, h2, W1l, W1r, b1.reshape(1, D), split_out=True)
  agg = _sc_aggregate(h2.reshape(2 * N_NODES, DH), src4, dst3, zf)
  h2 = _tc_layer(agg, cnt, h2, W2l, W2r, b2.reshape(1, D), split_out=True)
  agg = _sc_aggregate(h2.reshape(2 * N_NODES, DH), src4, dst3, zf)
  return _tc_layer(agg, cnt, h2, W3l, W3r, b3.reshape(1, D), split_out=False)


# speedup vs baseline: 3.0470x; 1.0968x over previous
import jax
import jax.numpy as jnp
from jax import lax
from jax.experimental import pallas as pl
from jax.experimental.pallas import tpu as pltpu
from jax.experimental.pallas import tpu_sc as plsc

N_NODES = 10000
N_EDGES = 160000
D = 256
DH = 128

NUM_TILES = 16
CHUNK = 128
CHUNKS_PER_TILE = 80
GCH = 8
GROUPS = CHUNKS_PER_TILE // GCH
EDGES_PER_TILE = CHUNK * CHUNKS_PER_TILE
E_PAD = EDGES_PER_TILE * NUM_TILES
N_PAD = 10112
ROWS_PER_TILE = N_PAD // NUM_TILES

def _mesh():
  return plsc.VectorSubcoreMesh(core_axis_name="c", subcore_axis_name="s")


def _sc_aggregate(h2, src4, dst3, zf):
  def body(h2_hbm, src_hbm, dst_hbm, zf_hbm, agg_hbm,
           accum, sbuf, dbuf, rows, sem):
    c = lax.axis_index("c")
    t = lax.axis_index("s")
    row0 = t * ROWS_PER_TILE
    pltpu.sync_copy(zf_hbm, accum.at[pl.ds(row0, ROWS_PER_TILE)])
    plsc.subcore_barrier()

    def group(g, carry):
      pltpu.sync_copy(src_hbm.at[c, t * GROUPS + g], sbuf)
      pltpu.sync_copy(dst_hbm.at[t * GROUPS + g], dbuf)

      def pair(p, carry2):
        j0 = 2 * p
        d0 = pltpu.async_copy(h2_hbm.at[sbuf.at[j0]],
                              rows.at[pl.ds(0, CHUNK)], sem)
        d1 = pltpu.async_copy(h2_hbm.at[sbuf.at[j0 + 1]],
                              rows.at[pl.ds(CHUNK, CHUNK)], sem)
        d0.wait()
        pltpu.sync_copy(rows.at[pl.ds(0, CHUNK)],
                        accum.at[dbuf.at[j0]], add=True)
        d1.wait()
        pltpu.sync_copy(rows.at[pl.ds(CHUNK, CHUNK)],
                        accum.at[dbuf.at[j0 + 1]], add=True)
        return carry2

      lax.fori_loop(0, GCH // 2, pair, 0)
      return carry

    lax.fori_loop(0, GROUPS, group, 0)
    plsc.subcore_barrier()
    pltpu.sync_copy(accum.at[pl.ds(row0, ROWS_PER_TILE)],
                    agg_hbm.at[c, pl.ds(row0, ROWS_PER_TILE)])

  fn = pl.kernel(
      body,
      out_type=jax.ShapeDtypeStruct((2, N_PAD, DH), jnp.float32),
      mesh=_mesh(),
      scratch_types=(pltpu.VMEM_SHARED((N_PAD, DH), jnp.float32),
                     pltpu.VMEM((GCH, CHUNK), jnp.int32),
                     pltpu.VMEM((GCH, CHUNK), jnp.int32),
                     pltpu.VMEM((2 * CHUNK, DH), jnp.float32),
                     pltpu.SemaphoreType.DMA))
  return fn(h2, src4, dst3, zf)


def _sc_count(dst3, zf, ones128):
  half = GROUPS // 2

  def body(dst_hbm, zf_hbm, ones_hbm, cnt_hbm, accum, dbuf, ones_v, sem):
    c = lax.axis_index("c")
    t = lax.axis_index("s")
    row0 = t * ROWS_PER_TILE
    pltpu.sync_copy(zf_hbm, accum.at[pl.ds(row0, ROWS_PER_TILE)])
    pltpu.sync_copy(ones_hbm, ones_v)
    plsc.subcore_barrier()

    def group(g, carry):
      pltpu.sync_copy(dst_hbm.at[t * GROUPS + c * half + g], dbuf)

      def chunk(j, carry2):
        pltpu.sync_copy(ones_v, accum.at[dbuf.at[j]], add=True)
        return carry2

      lax.fori_loop(0, GCH, chunk, 0)
      return carry

    lax.fori_loop(0, half, group, 0)
    plsc.subcore_barrier()
    pltpu.sync_copy(accum.at[pl.ds(row0, ROWS_PER_TILE)],
                    cnt_hbm.at[c, pl.ds(row0, ROWS_PER_TILE)])

  fn = pl.kernel(
      body,
      out_type=jax.ShapeDtypeStruct((2, N_PAD, DH), jnp.float32),
      mesh=_mesh(),
      scratch_types=(pltpu.VMEM_SHARED((N_PAD, DH), jnp.float32),
                     pltpu.VMEM((GCH, CHUNK), jnp.int32),
                     pltpu.VMEM((CHUNK, DH), jnp.float32),
                     pltpu.SemaphoreType.DMA))
  return fn(dst3, zf, ones128)


ROW_BLK = 400
GRID = N_NODES // ROW_BLK


def _tc_body(agg_ref, cnt_ref, h_ref, wl_ref, wr_ref, b_ref, *out_refs):
  agg = jnp.concatenate([agg_ref[0], agg_ref[1]], axis=1)
  cnt = cnt_ref[0][:, 0:1] + cnt_ref[1][:, 0:1]
  inv = 1.0 / jnp.maximum(cnt, 1.0)
  h = jnp.concatenate([h_ref[0], h_ref[1]], axis=1)
  out = (jnp.dot(agg * inv, wl_ref[...], preferred_element_type=jnp.float32)
         + jnp.dot(h, wr_ref[...], preferred_element_type=jnp.float32)
         + b_ref[...])
  act = jnp.where(out > 0, out, jnp.exp(jnp.minimum(out, 0.0)) - 1.0)
  if len(out_refs) == 1 and out_refs[0].shape[0] == 2:
    out_refs[0][0] = act[:, :DH]
    out_refs[0][1] = act[:, DH:]
  else:
    out_refs[0][...] = act


def _tc_layer(agg, cnt, h2, wl, wr, b, split_out):
  if split_out:
    out_shape = [jax.ShapeDtypeStruct((2, N_NODES, DH), jnp.float32)]
    out_specs = [pl.BlockSpec((2, ROW_BLK, DH), lambda r: (0, r, 0))]
  else:
    out_shape = [jax.ShapeDtypeStruct((N_NODES, D), jnp.float32)]
    out_specs = [pl.BlockSpec((ROW_BLK, D), lambda r: (r, 0))]
  res = pl.pallas_call(
      _tc_body,
      grid=(GRID,),
      in_specs=[
          pl.BlockSpec((2, ROW_BLK, DH), lambda r: (0, r, 0)),
          pl.BlockSpec((2, ROW_BLK, DH), lambda r: (0, r, 0)),
          pl.BlockSpec((2, ROW_BLK, DH), lambda r: (0, r, 0)),
          pl.BlockSpec((D, D), lambda r: (0, 0)),
          pl.BlockSpec((D, D), lambda r: (0, 0)),
          pl.BlockSpec((1, D), lambda r: (0, 0)),
      ],
      out_specs=out_specs,
      out_shape=out_shape,
  )(agg, cnt, h2, wl, wr, b)
  return res[0]


def kernel(x, edge_index, W1l, W1r, b1, W2l, W2r, b2, W3l, W3r, b3):
  x2 = jnp.stack([x[:, :DH], x[:, DH:]])
  src = edge_index[0]
  dst = edge_index[1]
  pad = E_PAD - N_EDGES
  src_p = jnp.concatenate([src, jnp.zeros((pad,), jnp.int32)])
  src4 = jnp.stack([src_p, src_p + N_NODES]).reshape(
      2, NUM_TILES * GROUPS, GCH, CHUNK)
  dst3 = jnp.concatenate([dst, jnp.full((pad,), N_NODES, jnp.int32)]).reshape(
      NUM_TILES * GROUPS, GCH, CHUNK)
  zf = jnp.zeros((ROWS_PER_TILE, DH), jnp.float32)
  ones128 = jnp.ones((CHUNK, DH), jnp.float32)

  cnt = _sc_count(dst3, zf, ones128)
  h2 = x2
  agg = _sc_aggregate(h2.reshape(2 * N_NODES, DH), src4, dst3, zf)
  h2 = _tc_layer(agg, cnt, h2, W1l, W1r, b1.reshape(1, D), split_out=True)
  agg = _sc_aggregate(h2.reshape(2 * N_NODES, DH), src4, dst3, zf)
  h2 = _tc_layer(agg, cnt, h2, W2l, W2r, b2.reshape(1, D), split_out=True)
  agg = _sc_aggregate(h2.reshape(2 * N_NODES, DH), src4, dst3, zf)
  return _tc_layer(agg, cnt, h2, W3l, W3r, b3.reshape(1, D), split_out=False)
